# baseline (speedup 1.0000x reference)
import functools

import jax
import jax.numpy as jnp
from jax import lax
from jax.experimental import pallas as pl
from jax.experimental.pallas import tpu as pltpu

HID = 4096
LAT = 32768
NTOK = 2048
K = 64
MAX_ALPHA = 3.0

BQ = 512


def _q_body(qv_ref, w_ref, q_ref):
    q_ref[...] = lax.dot_general(
        qv_ref[...], w_ref[...], (((1,), (1,)), ((), ())),
        preferred_element_type=jnp.float32)


def _compute_q(qv2, W):
    return pl.pallas_call(
        _q_body,
        grid=(HID // BQ,),
        in_specs=[
            pl.BlockSpec((1, HID), lambda i: (0, 0)),
            pl.BlockSpec((BQ, HID), lambda i: (i, 0)),
        ],
        out_specs=pl.BlockSpec((1, BQ), lambda i: (0, i)),
        out_shape=jax.ShapeDtypeStruct((1, HID), jnp.float32),
    )(qv2, W)


BF_A = 512


def _scores_body(q_ref, dw_ref, z_ref, ms_ref, act_ref):
    s = lax.dot_general(
        q_ref[...], dw_ref[...], (((1,), (0,)), ((), ())),
        preferred_element_type=jnp.float32)
    zmax = jnp.max(z_ref[...], axis=0, keepdims=True)
    act = (zmax > 0.0).astype(jnp.float32)
    ms_ref[...] = s - 1e9 * (1.0 - act)
    act_ref[...] = act


def _compute_scores(q, dw, z):
    return pl.pallas_call(
        _scores_body,
        grid=(LAT // BF_A,),
        in_specs=[
            pl.BlockSpec((1, HID), lambda j: (0, 0)),
            pl.BlockSpec((HID, BF_A), lambda j: (0, j)),
            pl.BlockSpec((NTOK, BF_A), lambda j: (0, j)),
        ],
        out_specs=[
            pl.BlockSpec((1, BF_A), lambda j: (0, j)),
            pl.BlockSpec((1, BF_A), lambda j: (0, j)),
        ],
        out_shape=[
            jax.ShapeDtypeStruct((1, LAT), jnp.float32),
            jax.ShapeDtypeStruct((1, LAT), jnp.float32),
        ],
    )(q, dw, z)


ROWS_B = 256
COLS_B = 128


def _topk_body(ms_ref, act_ref, ls_ref, alpha_ref):
    x = ms_ref[...]
    act = act_ref[...]
    lin = (lax.broadcasted_iota(jnp.int32, (ROWS_B, COLS_B), 0) * COLS_B
           + lax.broadcasted_iota(jnp.int32, (ROWS_B, COLS_B), 1))
    scale = jnp.minimum(jnp.exp(ls_ref[0, 0]), 10.0)

    def body(_, carry):
        x, alpha = carry
        m = jnp.max(x)
        idx = jnp.min(jnp.where(x == m, lin, jnp.int32(2 ** 30)))
        sel = lin == idx
        a_at = jnp.max(jnp.where(sel, act, -1.0))
        boost = 1.0 + (MAX_ALPHA - 1.0) / (1.0 + jnp.exp(-m * scale))
        alpha = jnp.where(sel & (a_at > 0.0), boost, alpha)
        x = jnp.where(sel, -jnp.inf, x)
        return x, alpha

    _, alpha = lax.fori_loop(0, K, body, (x, jnp.ones_like(x)))
    alpha_ref[...] = alpha


def _compute_alpha(ms, act, log_scale):
    return pl.pallas_call(
        _topk_body,
        in_specs=[
            pl.BlockSpec((ROWS_B, COLS_B), lambda: (0, 0)),
            pl.BlockSpec((ROWS_B, COLS_B), lambda: (0, 0)),
            pl.BlockSpec(memory_space=pltpu.SMEM),
        ],
        out_shape=jax.ShapeDtypeStruct((ROWS_B, COLS_B), jnp.float32),
    )(ms.reshape(ROWS_B, COLS_B), act.reshape(ROWS_B, COLS_B),
      log_scale.reshape(1, 1))


BF_C = 512


def _out_body(alpha_ref, z_ref, out_ref):
    a = alpha_ref[...]
    z = z_ref[...]
    out_ref[...] = jnp.where(z > 0.0, a, 1.0)


def _compute_out(alpha_row, z):
    return pl.pallas_call(
        _out_body,
        grid=(LAT // BF_C,),
        in_specs=[
            pl.BlockSpec((1, BF_C), lambda j: (0, j)),
            pl.BlockSpec((NTOK, BF_C), lambda j: (0, j)),
        ],
        out_specs=pl.BlockSpec((NTOK, BF_C), lambda j: (0, j)),
        out_shape=jax.ShapeDtypeStruct((NTOK, LAT), jnp.float32),
    )(alpha_row, z)


def kernel(question_vec, z, decoder_weight, W, log_scale):
    qv2 = question_vec.astype(jnp.float32).reshape(1, HID)
    q = _compute_q(qv2, W)
    ms, act = _compute_scores(q, decoder_weight, z)
    alpha = _compute_alpha(ms, act, log_scale)
    return _compute_out(alpha.reshape(1, LAT), z).astype(z.dtype)
